# Initial kernel scaffold; baseline (speedup 1.0000x reference)
import jax
import jax.numpy as jnp
from jax.experimental import pallas as pl


def kernel(x1, x2, edge_index_v1, edge_index_v2, batch, student_params, teacher_params, pred_params):
    raise NotImplementedError("write your pallas kernel here")


# speedup vs baseline: 2.6017x; 2.6017x over previous
import functools

import jax
import jax.numpy as jnp
from jax import lax
from jax.experimental import pallas as pl
from jax.experimental.pallas import tpu as pltpu
from jax.experimental.pallas import tpu_sc as plsc

_N = 10000
_E = 320000
_D = 128
_EMB = 256
_HID2 = 512
_R = 2000
_EPS = 1e-5
_NPAD = 10240


def _sc_segsum(h0, h1, src0, dst0, src1, dst1):
    C = 64
    NCH = _E // C
    CPT = -(-NCH // 16)
    NPT = _NPAD // 16
    RC = 32
    NRC = NPT // RC
    mesh = plsc.VectorSubcoreMesh(core_axis_name="c", subcore_axis_name="s")

    @functools.partial(
        pl.kernel,
        out_type=[jax.ShapeDtypeStruct((_NPAD, _D), jnp.float32),
                  jax.ShapeDtypeStruct((_NPAD, _D), jnp.float32),
                  jax.ShapeDtypeStruct((_NPAD, 16), jnp.float32),
                  jax.ShapeDtypeStruct((_NPAD, 16), jnp.float32)],
        mesh=mesh,
        scratch_types=[
            pltpu.VMEM_SHARED((_NPAD, _D), jnp.float32),
            pltpu.VMEM_SHARED((_NPAD, 16), jnp.float32),
            pltpu.VMEM((C,), jnp.int32),
            pltpu.VMEM((C,), jnp.int32),
            pltpu.VMEM((C, _D), jnp.float32),
            pltpu.VMEM((C, 16), jnp.float32),
            pltpu.VMEM((RC, _D), jnp.float32),
            pltpu.VMEM((RC, 16), jnp.float32),
            pltpu.VMEM((RC,), jnp.int32),
            pltpu.SemaphoreType.DMA,
        ],
    )
    def k(h0r, h1r, s0r, d0r, s1r, d1r,
          o0r, o1r, c0r, c1r, acc, cacc, gidx, sidx, rows, onev,
          stg, stg16, zidx, sem):
        c = lax.axis_index("c")
        s = lax.axis_index("s")
        nbase = s * NPT

        z16 = jnp.zeros((16,), jnp.float32)
        one16 = jnp.ones((16,), jnp.float32)

        @pl.loop(0, RC)
        def _(r):
            @pl.loop(0, _D // 16)
            def _(q):
                stg[r, pl.ds(q * 16, 16)] = z16

            stg16[r, pl.ds(0, 16)] = z16

        @pl.loop(0, C)
        def _(r):
            onev[r, pl.ds(0, 16)] = one16

        def set_zidx(j):
            for t in range(RC // 16):
                zidx[pl.ds(t * 16, 16)] = (
                    lax.iota(jnp.int32, 16) + (nbase + j * RC + t * 16))

        for j in range(NRC):
            set_zidx(j)
            pltpu.sync_copy(stg, acc.at[zidx])
            pltpu.sync_copy(stg16, cacc.at[zidx])

        plsc.subcore_barrier()

        def edges(h, se, de):
            @pl.loop(0, CPT)
            def _(i):
                k = s + i * 16

                @pl.when(k < NCH)
                def _():
                    off = k * C
                    pltpu.sync_copy(se.at[pl.ds(off, C)], gidx)
                    pltpu.sync_copy(de.at[pl.ds(off, C)], sidx)
                    pltpu.async_copy(h.at[gidx], rows, sem).wait()
                    pltpu.sync_copy(rows, acc.at[sidx], add=True)
                    pltpu.sync_copy(onev, cacc.at[sidx], add=True)

        @pl.when(c == 0)
        def _():
            edges(h0r, s0r, d0r)

        @pl.when(c == 1)
        def _():
            edges(h1r, s1r, d1r)

        plsc.subcore_barrier()

        def copyout(out, outc):
            for j in range(NRC):
                set_zidx(j)
                pltpu.async_copy(acc.at[zidx], stg, sem).wait()
                pltpu.sync_copy(stg, out.at[pl.ds(nbase + j * RC, RC)])
                pltpu.async_copy(cacc.at[zidx], stg16, sem).wait()
                pltpu.sync_copy(stg16, outc.at[pl.ds(nbase + j * RC, RC)])

        @pl.when(c == 0)
        def _():
            copyout(o0r, c0r)

        @pl.when(c == 1)
        def _():
            copyout(o1r, c1r)

    o0, o1, c0, c1 = k(h0, h1, src0, dst0, src1, dst1)
    return o0[:_N], o1[:_N], c0[:_N], c1[:_N]


def _mlp(z, w1, b1, w2, b2):
    zz = jnp.maximum(jnp.dot(z, w1, preferred_element_type=jnp.float32) + b1, 0.0)
    return jnp.maximum(jnp.dot(zz, w2, preferred_element_type=jnp.float32) + b2, 0.0)


def _acc_out(i, ref, val):
    @pl.when(i == 0)
    def _():
        ref[...] = val

    @pl.when(i > 0)
    def _():
        ref[...] = ref[...] + val


def _stats8(a, b):
    return jnp.concatenate(
        [jnp.sum(a, 0)[None], jnp.sum(a * a, 0)[None],
         jnp.sum(b, 0)[None], jnp.sum(b * b, 0)[None],
         jnp.zeros((4, a.shape[1]), jnp.float32)], 0)


def _k1_body(x_ref, ac_ref, w1s, b1s, w2s, b2s, w1t, b1t, w2t, b2t,
             hs_ref, ht_ref, st_ref):
    i = pl.program_id(0)
    z = x_ref[...] + ac_ref[...]
    hs = _mlp(z, w1s[...], b1s[...], w2s[...], b2s[...])
    ht = _mlp(z, w1t[...], b1t[...], w2t[...], b2t[...])
    hs_ref[...] = hs
    ht_ref[...] = ht
    _acc_out(i, st_ref, _stats8(hs, ht))


def _wspec():
    return pl.BlockSpec((_D, _D), lambda i: (0, 0))


def _bspec():
    return pl.BlockSpec((1, _D), lambda i: (0, 0))


def _rspec(w=_D):
    return pl.BlockSpec((_R, w), lambda i: (i, 0))


def _k1(x, aggcnt, ps, pt):
    return pl.pallas_call(
        _k1_body,
        grid=(_N // _R,),
        in_specs=[_rspec(), _rspec(),
                  _wspec(), _bspec(), _wspec(), _bspec(),
                  _wspec(), _bspec(), _wspec(), _bspec()],
        out_specs=[_rspec(), _rspec(), pl.BlockSpec((8, _D), lambda i: (0, 0))],
        out_shape=[jax.ShapeDtypeStruct((_N, _D), jnp.float32),
                   jax.ShapeDtypeStruct((_N, _D), jnp.float32),
                   jax.ShapeDtypeStruct((8, _D), jnp.float32)],
    )(x, aggcnt,
      ps['w1'], ps['b1'].reshape(1, -1), ps['w2'], ps['b2'].reshape(1, -1),
      pt['w1'], pt['b1'].reshape(1, -1), pt['w2'], pt['b2'].reshape(1, -1))


def _k2_body(hs_ref, ht_ref, ags_ref, agt_ref, cnt_ref,
             ss_ref, ts_ref, st_ref_a, tt_ref,
             w1s, b1s, w2s, b2s, w1t, b1t, w2t, b2t,
             hs2_ref, ht2_ref, st_ref):
    i = pl.program_id(0)
    cnt1 = 1.0 + cnt_ref[:, :1]
    zs = ss_ref[...] * (hs_ref[...] + ags_ref[...]) + cnt1 * ts_ref[...]
    zt = st_ref_a[...] * (ht_ref[...] + agt_ref[...]) + cnt1 * tt_ref[...]
    hs2 = _mlp(zs, w1s[...], b1s[...], w2s[...], b2s[...])
    ht2 = _mlp(zt, w1t[...], b1t[...], w2t[...], b2t[...])
    hs2_ref[...] = hs2
    ht2_ref[...] = ht2
    _acc_out(i, st_ref, _stats8(hs2, ht2))


def _k2(hs, ht, ags, agt, cnt16, aff_s, aff_t, ps, pt):
    return pl.pallas_call(
        _k2_body,
        grid=(_N // _R,),
        in_specs=[_rspec(), _rspec(), _rspec(), _rspec(),
                  pl.BlockSpec((_R, 16), lambda i: (i, 0)),
                  _bspec(), _bspec(), _bspec(), _bspec(),
                  _wspec(), _bspec(), _wspec(), _bspec(),
                  _wspec(), _bspec(), _wspec(), _bspec()],
        out_specs=[_rspec(), _rspec(), pl.BlockSpec((8, _D), lambda i: (0, 0))],
        out_shape=[jax.ShapeDtypeStruct((_N, _D), jnp.float32),
                   jax.ShapeDtypeStruct((_N, _D), jnp.float32),
                   jax.ShapeDtypeStruct((8, _D), jnp.float32)],
    )(hs, ht, ags, agt, cnt16,
      aff_s[0], aff_s[1], aff_t[0], aff_t[1],
      ps['w1'], ps['b1'].reshape(1, -1), ps['w2'], ps['b2'].reshape(1, -1),
      pt['w1'], pt['b1'].reshape(1, -1), pt['w2'], pt['b2'].reshape(1, -1))


def _k3_body(h1_ref, h2_ref, s1_ref, t1_ref, s2_ref, t2_ref,
             w1a, w1b, b1, z_ref, st_ref):
    i = pl.program_id(0)
    u1 = s1_ref[...] * h1_ref[...] + t1_ref[...]
    u2 = s2_ref[...] * h2_ref[...] + t2_ref[...]
    z = (jnp.dot(u1, w1a[...], preferred_element_type=jnp.float32)
         + jnp.dot(u2, w1b[...], preferred_element_type=jnp.float32)
         + b1[...])
    z_ref[...] = z
    st = jnp.concatenate(
        [jnp.sum(z, 0)[None], jnp.sum(z * z, 0)[None],
         jnp.zeros((6, _HID2), jnp.float32)], 0)
    _acc_out(i, st_ref, st)


def _k3(h1, h2, aff1, aff2, pp):
    w1 = pp['w1']
    return pl.pallas_call(
        _k3_body,
        grid=(_N // _R,),
        in_specs=[_rspec(), _rspec(),
                  _bspec(), _bspec(), _bspec(), _bspec(),
                  pl.BlockSpec((_D, _HID2), lambda i: (0, 0)),
                  pl.BlockSpec((_D, _HID2), lambda i: (0, 0)),
                  pl.BlockSpec((1, _HID2), lambda i: (0, 0))],
        out_specs=[_rspec(_HID2), pl.BlockSpec((8, _HID2), lambda i: (0, 0))],
        out_shape=[jax.ShapeDtypeStruct((_N, _HID2), jnp.float32),
                   jax.ShapeDtypeStruct((8, _HID2), jnp.float32)],
    )(h1, h2, aff1[0], aff1[1], aff2[0], aff2[1],
      w1[:_D], w1[_D:], pp['b1'].reshape(1, -1))


def _k4_body(z1_ref, z2_ref, h1t1, h2t1, h1t2, h2t2,
             sp1_ref, tp1_ref, sp2_ref, tp2_ref, a_ref,
             s1v1, t1v1, s2v1, t2v1, s1v2, t1v2, s2v2, t2v2,
             w2_ref, b2_ref, out_ref):
    i = pl.program_id(0)
    a = a_ref[0, 0]

    def pred(zr, spr, tpr):
        u = spr[...] * zr[...] + tpr[...]
        u = jnp.where(u > 0, u, a * u)
        return jnp.dot(u, w2_ref[...], preferred_element_type=jnp.float32) + b2_ref[...]

    p1 = pred(z1_ref, sp1_ref, tp1_ref)
    p2 = pred(z2_ref, sp2_ref, tp2_ref)
    tv1 = jnp.concatenate([s1v1[...] * h1t1[...] + t1v1[...],
                           s2v1[...] * h2t1[...] + t2v1[...]], axis=1)
    tv2 = jnp.concatenate([s1v2[...] * h1t2[...] + t1v2[...],
                           s2v2[...] * h2t2[...] + t2v2[...]], axis=1)

    def cos(p, t):
        d = jnp.sum(p * t, 1)
        n1 = jnp.maximum(jnp.sqrt(jnp.sum(p * p, 1)), 1e-8)
        n2 = jnp.maximum(jnp.sqrt(jnp.sum(t * t, 1)), 1e-8)
        return d / (n1 * n2)

    bsum = jnp.sum(4.0 - 2.0 * cos(p1, tv2) - 2.0 * cos(p2, tv1))
    _acc_out(i, out_ref, jnp.full((8, 128), bsum, jnp.float32))


def _k4(z1, z2, t_raw1, t_raw2, aff_p1, aff_p2, a, t_aff1, t_aff2, pp):
    return pl.pallas_call(
        _k4_body,
        grid=(_N // _R,),
        in_specs=[_rspec(_HID2), _rspec(_HID2),
                  _rspec(), _rspec(), _rspec(), _rspec(),
                  pl.BlockSpec((1, _HID2), lambda i: (0, 0)),
                  pl.BlockSpec((1, _HID2), lambda i: (0, 0)),
                  pl.BlockSpec((1, _HID2), lambda i: (0, 0)),
                  pl.BlockSpec((1, _HID2), lambda i: (0, 0)),
                  pl.BlockSpec(memory_space=pltpu.SMEM),
                  _bspec(), _bspec(), _bspec(), _bspec(),
                  _bspec(), _bspec(), _bspec(), _bspec(),
                  pl.BlockSpec((_HID2, _EMB), lambda i: (0, 0)),
                  pl.BlockSpec((1, _EMB), lambda i: (0, 0))],
        out_specs=pl.BlockSpec((8, 128), lambda i: (0, 0)),
        out_shape=jax.ShapeDtypeStruct((8, 128), jnp.float32),
    )(z1, z2, t_raw1[0], t_raw1[1], t_raw2[0], t_raw2[1],
      aff_p1[0], aff_p1[1], aff_p2[0], aff_p2[1], a.reshape(1, 1),
      t_aff1[0][0], t_aff1[0][1], t_aff1[1][0], t_aff1[1][1],
      t_aff2[0][0], t_aff2[0][1], t_aff2[1][0], t_aff2[1][1],
      pp['w2'], pp['b2'].reshape(1, -1))


def _affine(s, sq, g, b):
    mu = s / _N
    var = sq / _N - mu * mu
    sc = g / jnp.sqrt(var + _EPS)
    return (sc.reshape(1, -1), (b - mu * sc).reshape(1, -1))


def kernel(x1, x2, edge_index_v1, edge_index_v2, batch,
           student_params, teacher_params, pred_params):
    del batch
    src1 = edge_index_v1[0].astype(jnp.int32)
    dst1 = edge_index_v1[1].astype(jnp.int32)
    src2 = edge_index_v2[0].astype(jnp.int32)
    dst2 = edge_index_v2[1].astype(jnp.int32)

    agg1, agg2, cnt16_1, cnt16_2 = _sc_segsum(x1, x2, src1, dst1, src2, dst2)

    def enc_view(x, agg0, cnt16, src, dst):
        hs1, ht1, st1 = _k1(x, agg0, student_params[0], teacher_params[0])
        aff_s1 = _affine(st1[0], st1[1], student_params[0]['bn_g'], student_params[0]['bn_b'])
        aff_t1 = _affine(st1[2], st1[3], teacher_params[0]['bn_g'], teacher_params[0]['bn_b'])
        ags, agt, _, _ = _sc_segsum(hs1, ht1, src, dst, src, dst)
        hs2, ht2, st2 = _k2(hs1, ht1, ags, agt, cnt16, aff_s1, aff_t1,
                            student_params[1], teacher_params[1])
        aff_s2 = _affine(st2[0], st2[1], student_params[1]['bn_g'], student_params[1]['bn_b'])
        aff_t2 = _affine(st2[2], st2[3], teacher_params[1]['bn_g'], teacher_params[1]['bn_b'])
        return ((hs1, hs2), (aff_s1, aff_s2)), ((ht1, ht2), (aff_t1, aff_t2))

    (s_raw1, s_aff1), (t_raw1, t_aff1) = enc_view(x1, agg1, cnt16_1, src1, dst1)
    (s_raw2, s_aff2), (t_raw2, t_aff2) = enc_view(x2, agg2, cnt16_2, src2, dst2)

    z1, st3_1 = _k3(s_raw1[0], s_raw1[1], s_aff1[0], s_aff1[1], pred_params)
    z2, st3_2 = _k3(s_raw2[0], s_raw2[1], s_aff2[0], s_aff2[1], pred_params)
    aff_p1 = _affine(st3_1[0], st3_1[1], pred_params['bn_g'], pred_params['bn_b'])
    aff_p2 = _affine(st3_2[0], st3_2[1], pred_params['bn_g'], pred_params['bn_b'])

    out1 = _k4(z1, z2, t_raw1, t_raw2, aff_p1, aff_p2, pred_params['a'],
               t_aff1, t_aff2, pred_params)
    return out1[0, 0] / _N
